# baseline (speedup 1.0000x reference)
import functools

import jax
import jax.numpy as jnp
from jax import lax
from jax.experimental import pallas as pl
from jax.experimental.pallas import tpu as pltpu
from jax.experimental.pallas import tpu_sc as plsc

N = 100000
E = 1600000
DI = 9
H = 64
KC = 1024

NC = 2
NS = 16
NW = NC * NS
EPT = E // NW
CH = 1000
NCH = EPT // CH
CH2 = 400
NCH2 = EPT // CH2

CH3 = 2000
NCH3 = EPT // CH3
ROWS2 = 1032
DUMMY = 1024
CAP = 144
FLUSH_AT = 128

BLK = 4000
GRID = N // BLK

_mesh = plsc.VectorSubcoreMesh(core_axis_name="c", subcore_axis_name="s")
_sc_params = pltpu.CompilerParams(use_tc_tiling_on_sc=False,
                                 needs_layout_passes=False)


def _k1_body(dst_hbm, zeros16_hbm, ones_hbm, deg_out, dstv2, onesv, acc_sh,
             ss0, ss1):
    cid = lax.axis_index("c")
    sid = lax.axis_index("s")
    wid = cid * NS + sid
    base = wid * EPT
    ss = (ss0, ss1)

    @pl.when(sid == 0)
    def _():
        pltpu.sync_copy(zeros16_hbm, acc_sh)

    pltpu.sync_copy(ones_hbm, onesv)
    plsc.subcore_barrier()

    def load_idx(k, b):
        pltpu.sync_copy(dst_hbm.at[pl.ds(base + k * CH, CH)], dstv2.at[b])

    def post_scatter(b):
        pltpu.async_copy(onesv, acc_sh.at[dstv2.at[b]], ss[b], add=True)

    def wait_scatter(b):
        pltpu.make_async_copy(onesv, acc_sh.at[dstv2.at[b]], ss[b]).wait()

    for b in range(2):
        load_idx(b, b)
        post_scatter(b)

    @pl.loop(1, NCH // 2)
    def _(j):
        for b in range(2):
            k = 2 * j + b
            wait_scatter(b)
            load_idx(k, b)
            post_scatter(b)

    wait_scatter(0)
    wait_scatter(1)
    plsc.subcore_barrier()

    @pl.when(sid == 0)
    def _():
        pltpu.sync_copy(acc_sh, deg_out.at[cid])


_k1 = functools.partial(
    pl.kernel,
    out_type=jax.ShapeDtypeStruct((NC, N, 16), jnp.float32),
    mesh=_mesh,
    compiler_params=_sc_params,
    scratch_types=[
        pltpu.VMEM((2, CH), jnp.int32),
        pltpu.VMEM((CH, 16), jnp.float32),
        pltpu.VMEM_SHARED((N, 16), jnp.float32),
        pltpu.SemaphoreType.DMA,
        pltpu.SemaphoreType.DMA,
    ],
)(_k1_body)


def _k2_body(src_hbm, dst_hbm, xs_hbm, zeros16_hbm, acc_out,
             srcv2, dstv2, rowv2, acc_sh, ss0, ss1):
    cid = lax.axis_index("c")
    sid = lax.axis_index("s")
    wid = cid * NS + sid
    base = wid * EPT
    ss = (ss0, ss1)

    @pl.when(sid == 0)
    def _():
        pltpu.sync_copy(zeros16_hbm, acc_sh)

    plsc.subcore_barrier()

    def do_chunk(k, b):
        pltpu.sync_copy(src_hbm.at[pl.ds(base + k * CH2, CH2)], srcv2.at[b])
        pltpu.sync_copy(dst_hbm.at[pl.ds(base + k * CH2, CH2)], dstv2.at[b])
        pltpu.sync_copy(xs_hbm.at[srcv2.at[b]], rowv2.at[b])
        pltpu.async_copy(rowv2.at[b], acc_sh.at[dstv2.at[b]], ss[b], add=True)

    def wait_scatter(b):
        pltpu.make_async_copy(rowv2.at[b], acc_sh.at[dstv2.at[b]], ss[b]).wait()

    for b in range(2):
        do_chunk(b, b)

    @pl.loop(1, (NCH2 - 1) // 2)
    def _(j):
        for b in range(2):
            wait_scatter(b)
            do_chunk(2 * j + b, b)

    wait_scatter(0)
    do_chunk(NCH2 - 1, 0)
    wait_scatter(0)
    wait_scatter(1)
    plsc.subcore_barrier()

    @pl.when(sid == 0)
    def _():
        pltpu.sync_copy(acc_sh, acc_out.at[cid])


_k2 = functools.partial(
    pl.kernel,
    out_type=jax.ShapeDtypeStruct((NC, N, 16), jnp.float32),
    mesh=_mesh,
    compiler_params=_sc_params,
    scratch_types=[
        pltpu.VMEM((2, CH2), jnp.int32),
        pltpu.VMEM((2, CH2), jnp.int32),
        pltpu.VMEM((2, CH2, 16), jnp.float32),
        pltpu.VMEM_SHARED((N, 16), jnp.float32),
        pltpu.SemaphoreType.DMA,
        pltpu.SemaphoreType.DMA,
    ],
)(_k2_body)


def _make_k3():
    def body(src_hbm, dst_hbm, cand_hbm, rep_hbm, h1s_hbm, dinv16_hbm,
             negones_hbm, zeros2_hbm,
             acc2_out, candrows_out, canddinv_out,
             candv, repv, srcv, dstv, csrc, cslot, rowbuf,
             kidx, crow, cdv, map_ref, acc2_sh):
        cid = lax.axis_index("c")
        sid = lax.axis_index("s")
        wid = cid * NS + sid

        pltpu.sync_copy(cand_hbm, candv)
        pltpu.sync_copy(rep_hbm, repv)

        @pl.when(sid == 0)
        def _():
            pltpu.sync_copy(zeros2_hbm, acc2_sh)

        pltpu.sync_copy(negones_hbm, map_ref)

        @pl.loop(0, KC, step=16)
        def _(g):
            c16 = candv[pl.ds(g, 16)]
            r16 = repv[pl.ds(g, 16)]
            plsc.store_scatter(map_ref, [c16], r16)

        plsc.subcore_barrier()

        def reset_bufs():
            for t in range(CAP // 16):
                csrc[pl.ds(t * 16, 16)] = jnp.zeros((16,), jnp.int32)
                cslot[pl.ds(t * 16, 16)] = jnp.full((16,), DUMMY, jnp.int32)

        reset_bufs()

        def flush():
            pltpu.sync_copy(h1s_hbm.at[csrc], rowbuf)
            pltpu.sync_copy(rowbuf, acc2_sh.at[cslot], add=True)
            reset_bufs()

        def grp(g, w):
            d16 = dstv[pl.ds(g * 16, 16)]
            s16 = srcv[pl.ds(g * 16, 16)]
            sl16 = plsc.load_gather(map_ref, [d16])
            m = sl16 >= 0
            plsc.store_compressed(csrc.at[pl.ds(w, 16)], s16, mask=m)
            plsc.store_compressed(cslot.at[pl.ds(w, 16)], sl16, mask=m)
            cnt = jnp.sum(jnp.where(m, 1, 0).astype(jnp.int32))
            w2 = w + cnt

            @pl.when(w2 >= FLUSH_AT)
            def _():
                flush()

            return jnp.where(w2 >= FLUSH_AT, 0, w2)

        def chunk(ci, w):
            off = wid * EPT + ci * CH3
            pltpu.sync_copy(src_hbm.at[pl.ds(off, CH3)], srcv)
            pltpu.sync_copy(dst_hbm.at[pl.ds(off, CH3)], dstv)
            return lax.fori_loop(0, CH3 // 16, grp, w)

        lax.fori_loop(0, NCH3, chunk, jnp.int32(0))
        flush()

        kbase = wid * (KC // NW)
        kidx[pl.ds(0, 16)] = candv[pl.ds(kbase, 16)]
        kidx[pl.ds(16, 16)] = candv[pl.ds(kbase + 16, 16)]
        pltpu.sync_copy(h1s_hbm.at[kidx], crow)
        pltpu.sync_copy(dinv16_hbm.at[kidx], cdv)
        pltpu.sync_copy(crow, candrows_out.at[pl.ds(kbase, KC // NW)])
        pltpu.sync_copy(cdv, canddinv_out.at[pl.ds(kbase, KC // NW)])

        plsc.subcore_barrier()

        @pl.when(sid == 0)
        def _():
            pltpu.sync_copy(acc2_sh, acc2_out.at[cid])

    return functools.partial(
        pl.kernel,
        out_type=[
            jax.ShapeDtypeStruct((NC, ROWS2, H), jnp.float32),
            jax.ShapeDtypeStruct((KC, H), jnp.float32),
            jax.ShapeDtypeStruct((KC, 16), jnp.float32),
        ],
        mesh=_mesh,
        compiler_params=_sc_params,
        scratch_types=[
            pltpu.VMEM((KC,), jnp.int32),
            pltpu.VMEM((KC,), jnp.int32),
            pltpu.VMEM((CH3,), jnp.int32),
            pltpu.VMEM((CH3,), jnp.int32),
            pltpu.VMEM((CAP,), jnp.int32),
            pltpu.VMEM((CAP,), jnp.int32),
            pltpu.VMEM((CAP, H), jnp.float32),
            pltpu.VMEM((KC // NW,), jnp.int32),
            pltpu.VMEM((KC // NW, H), jnp.float32),
            pltpu.VMEM((KC // NW, 16), jnp.float32),
            pltpu.VMEM((N,), jnp.int32),
            pltpu.VMEM_SHARED((ROWS2, H), jnp.float32),
        ],
    )(body)


_k3 = _make_k3()


def _t0_body(col_ref, row_ref, rep_ref):
    col = col_ref[...]
    row = row_ref[...]
    eq = col == row
    jcol = lax.broadcasted_iota(jnp.int32, (KC, KC), 1)
    masked = jnp.where(eq, jcol, jnp.int32(1 << 30))
    rep_ref[...] = jnp.min(masked, axis=1, keepdims=True)


_t0 = pl.pallas_call(
    _t0_body,
    out_shape=jax.ShapeDtypeStruct((KC, 1), jnp.int32),
)


def _t1_body(deg_ref, x_ref, xs_ref, dinv16_ref):
    deg = deg_ref[0, :, 0:1] + deg_ref[1, :, 0:1] + 1.0
    dinv = 1.0 / jnp.sqrt(deg)
    dinv16_ref[...] = jnp.broadcast_to(dinv, (BLK, 16))
    xs = x_ref[...] * dinv
    xs_ref[...] = jnp.concatenate(
        [xs, jnp.zeros((BLK, 16 - DI), jnp.float32)], axis=1)


_t1 = pl.pallas_call(
    _t1_body,
    grid=(GRID,),
    in_specs=[
        pl.BlockSpec((NC, BLK, 16), lambda i: (0, i, 0)),
        pl.BlockSpec((BLK, DI), lambda i: (i, 0)),
    ],
    out_specs=[
        pl.BlockSpec((BLK, 16), lambda i: (i, 0)),
        pl.BlockSpec((BLK, 16), lambda i: (i, 0)),
    ],
    out_shape=[
        jax.ShapeDtypeStruct((N, 16), jnp.float32),
        jax.ShapeDtypeStruct((N, 16), jnp.float32),
    ],
)


def _t2_body(acc_ref, x_ref, dinv16_ref, w1_ref, b1_ref, h1s_ref):
    dinv = dinv16_ref[:, 0:1]
    acc9 = acc_ref[0, :, 0:DI] + acc_ref[1, :, 0:DI]
    agg = acc9 * dinv + x_ref[...] * (dinv * dinv)
    h1 = jnp.dot(agg, w1_ref[...], preferred_element_type=jnp.float32)
    h1 = jnp.maximum(h1 + b1_ref[...], 0.0)
    h1s_ref[...] = h1 * dinv


_t2 = pl.pallas_call(
    _t2_body,
    grid=(GRID,),
    in_specs=[
        pl.BlockSpec((NC, BLK, 16), lambda i: (0, i, 0)),
        pl.BlockSpec((BLK, DI), lambda i: (i, 0)),
        pl.BlockSpec((BLK, 16), lambda i: (i, 0)),
        pl.BlockSpec((DI, H), lambda i: (0, 0)),
        pl.BlockSpec((1, H), lambda i: (0, 0)),
    ],
    out_specs=pl.BlockSpec((BLK, H), lambda i: (i, 0)),
    out_shape=jax.ShapeDtypeStruct((N, H), jnp.float32),
)


def _t3_body(acc2_ref, rep_ref, candrows_ref, canddinv_ref,
             w2_ref, b2_ref, ws1_ref, bs1_ref, ws2_ref, bs2_ref, out_ref):
    total = acc2_ref[0] + acc2_ref[1]
    rep = rep_ref[...]
    jrow = lax.broadcasted_iota(jnp.int32, (KC, ROWS2), 1)
    onehot = jnp.where(rep == jrow, 1.0, 0.0)
    cand_acc = jnp.dot(onehot, total, preferred_element_type=jnp.float32)
    dinv = canddinv_ref[:, 0:1]
    pre = dinv * (cand_acc + candrows_ref[...])
    out2 = jnp.dot(pre, w2_ref[...],
                   preferred_element_type=jnp.float32) + b2_ref[...]
    cand_h = jnp.maximum(out2, 0.0)
    t = jnp.tanh(jnp.dot(cand_h, ws1_ref[...],
                         preferred_element_type=jnp.float32) + bs1_ref[...])
    out_ref[...] = jnp.dot(t, ws2_ref[...],
                           preferred_element_type=jnp.float32) + bs2_ref[...]


_t3 = pl.pallas_call(
    _t3_body,
    out_shape=jax.ShapeDtypeStruct((KC, 1), jnp.float32),
)


def kernel(x, edge_index, candidate_indices, W1, b1, W2, b2, Ws1, bs1, Ws2, bs2):
    src = edge_index[0]
    dst = edge_index[1]
    cand = candidate_indices.astype(jnp.int32)

    zeros16 = jnp.zeros((N, 16), jnp.float32)
    ones_rows = jnp.ones((CH, 16), jnp.float32)
    negones = jnp.full((N,), -1, jnp.int32)
    zeros2 = jnp.zeros((ROWS2, H), jnp.float32)

    rep = _t0(cand.reshape(KC, 1), cand.reshape(1, KC))

    deg2 = _k1(dst, zeros16, ones_rows)
    xs16, dinv16 = _t1(deg2, x)
    acc1 = _k2(src, dst, xs16, zeros16)
    h1s = _t2(acc1, x, dinv16, W1, b1.reshape(1, H))
    acc2, candrows, canddinv = _k3(
        src, dst, cand, rep.reshape(KC), h1s, dinv16, negones, zeros2)
    scores = _t3(acc2, rep, candrows, canddinv,
                 W2, b2.reshape(1, H), Ws1, bs1.reshape(1, H),
                 Ws2, bs2.reshape(1, 1))
    return scores.reshape(KC)


# speedup vs baseline: 1.1040x; 1.1040x over previous
import functools

import jax
import jax.numpy as jnp
from jax import lax
from jax.experimental import pallas as pl
from jax.experimental.pallas import tpu as pltpu
from jax.experimental.pallas import tpu_sc as plsc

N = 100000
E = 1600000
DI = 9
H = 64
KC = 1024

NC = 2
NS = 16
NW = NC * NS
EPT = E // NW
CH = 1000
NCH = EPT // CH

CH3 = 2000
NCH3 = EPT // CH3
ROWS2 = 1032
DUMMY = 1024
CAP = 144
FLUSH_AT = 128

BLK = 4000
GRID = N // BLK

_mesh = plsc.VectorSubcoreMesh(core_axis_name="c", subcore_axis_name="s")
_sc_params = pltpu.CompilerParams(use_tc_tiling_on_sc=False,
                                 needs_layout_passes=False)


def _k1_body(ei_hbm, zeros16_hbm, ones_hbm, deg_out, dstv, onesv, acc_sh):
    cid = lax.axis_index("c")
    sid = lax.axis_index("s")
    wid = cid * NS + sid

    @pl.when(sid == 0)
    def _():
        pltpu.sync_copy(zeros16_hbm, acc_sh)

    pltpu.sync_copy(ones_hbm, onesv)
    plsc.subcore_barrier()

    @pl.loop(0, NCH)
    def _(ci):
        off = wid * EPT + ci * CH
        pltpu.sync_copy(ei_hbm.at[1, pl.ds(off, CH)], dstv)
        pltpu.sync_copy(onesv, acc_sh.at[dstv], add=True)

    plsc.subcore_barrier()

    @pl.when(sid == 0)
    def _():
        pltpu.sync_copy(acc_sh, deg_out.at[cid])


_k1 = functools.partial(
    pl.kernel,
    out_type=jax.ShapeDtypeStruct((NC, N, 16), jnp.float32),
    mesh=_mesh,
    compiler_params=_sc_params,
    scratch_types=[
        pltpu.VMEM((CH,), jnp.int32),
        pltpu.VMEM((CH, 16), jnp.float32),
        pltpu.VMEM_SHARED((N, 16), jnp.float32),
    ],
)(_k1_body)


def _k2_body(ei_hbm, xs_hbm, zeros16_hbm, acc_out,
             srcv, dstv, rowv, acc_sh):
    cid = lax.axis_index("c")
    sid = lax.axis_index("s")
    wid = cid * NS + sid

    @pl.when(sid == 0)
    def _():
        pltpu.sync_copy(zeros16_hbm, acc_sh)

    plsc.subcore_barrier()

    @pl.loop(0, NCH)
    def _(ci):
        off = wid * EPT + ci * CH
        pltpu.sync_copy(ei_hbm.at[0, pl.ds(off, CH)], srcv)
        pltpu.sync_copy(ei_hbm.at[1, pl.ds(off, CH)], dstv)
        pltpu.sync_copy(xs_hbm.at[srcv], rowv)
        pltpu.sync_copy(rowv, acc_sh.at[dstv], add=True)

    plsc.subcore_barrier()

    @pl.when(sid == 0)
    def _():
        pltpu.sync_copy(acc_sh, acc_out.at[cid])


_k2 = functools.partial(
    pl.kernel,
    out_type=jax.ShapeDtypeStruct((NC, N, 16), jnp.float32),
    mesh=_mesh,
    compiler_params=_sc_params,
    scratch_types=[
        pltpu.VMEM((CH,), jnp.int32),
        pltpu.VMEM((CH,), jnp.int32),
        pltpu.VMEM((CH, 16), jnp.float32),
        pltpu.VMEM_SHARED((N, 16), jnp.float32),
    ],
)(_k2_body)


def _make_k3():
    def body(ei_hbm, cand_hbm, rep_hbm, h1s_hbm, dinv16_hbm,
             negones_hbm, zeros2_hbm,
             acc2_out, candrows_out, canddinv_out,
             candv, repv, srcv, dstv, csrc, cslot, rowbuf,
             kidx, crow, cdv, map_ref, acc2_sh):
        cid = lax.axis_index("c")
        sid = lax.axis_index("s")
        wid = cid * NS + sid

        pltpu.sync_copy(cand_hbm, candv)
        pltpu.sync_copy(rep_hbm, repv)

        @pl.when(sid == 0)
        def _():
            pltpu.sync_copy(zeros2_hbm, acc2_sh)

        pltpu.sync_copy(negones_hbm, map_ref)

        @pl.loop(0, KC, step=16)
        def _(g):
            c16 = candv[pl.ds(g, 16)]
            r16 = repv[pl.ds(g, 16)]
            plsc.store_scatter(map_ref, [c16], r16)

        plsc.subcore_barrier()

        def reset_bufs():
            for t in range(CAP // 16):
                csrc[pl.ds(t * 16, 16)] = jnp.zeros((16,), jnp.int32)
                cslot[pl.ds(t * 16, 16)] = jnp.full((16,), DUMMY, jnp.int32)

        reset_bufs()

        def flush():
            pltpu.sync_copy(h1s_hbm.at[csrc], rowbuf)
            pltpu.sync_copy(rowbuf, acc2_sh.at[cslot], add=True)
            reset_bufs()

        def grp(g, w):
            d16 = dstv[pl.ds(g * 16, 16)]
            s16 = srcv[pl.ds(g * 16, 16)]
            sl16 = plsc.load_gather(map_ref, [d16])
            m = sl16 >= 0
            plsc.store_compressed(csrc.at[pl.ds(w, 16)], s16, mask=m)
            plsc.store_compressed(cslot.at[pl.ds(w, 16)], sl16, mask=m)
            cnt = jnp.sum(jnp.where(m, 1, 0).astype(jnp.int32))
            w2 = w + cnt

            @pl.when(w2 >= FLUSH_AT)
            def _():
                flush()

            return jnp.where(w2 >= FLUSH_AT, 0, w2)

        def chunk(ci, w):
            off = wid * EPT + ci * CH3
            pltpu.sync_copy(ei_hbm.at[0, pl.ds(off, CH3)], srcv)
            pltpu.sync_copy(ei_hbm.at[1, pl.ds(off, CH3)], dstv)
            return lax.fori_loop(0, CH3 // 16, grp, w)

        lax.fori_loop(0, NCH3, chunk, jnp.int32(0))
        flush()

        kbase = wid * (KC // NW)
        kidx[pl.ds(0, 16)] = candv[pl.ds(kbase, 16)]
        kidx[pl.ds(16, 16)] = candv[pl.ds(kbase + 16, 16)]
        pltpu.sync_copy(h1s_hbm.at[kidx], crow)
        pltpu.sync_copy(dinv16_hbm.at[kidx], cdv)
        pltpu.sync_copy(crow, candrows_out.at[pl.ds(kbase, KC // NW)])
        pltpu.sync_copy(cdv, canddinv_out.at[pl.ds(kbase, KC // NW)])

        plsc.subcore_barrier()

        @pl.when(sid == 0)
        def _():
            pltpu.sync_copy(acc2_sh, acc2_out.at[cid])

    return functools.partial(
        pl.kernel,
        out_type=[
            jax.ShapeDtypeStruct((NC, ROWS2, H), jnp.float32),
            jax.ShapeDtypeStruct((KC, H), jnp.float32),
            jax.ShapeDtypeStruct((KC, 16), jnp.float32),
        ],
        mesh=_mesh,
        compiler_params=_sc_params,
        scratch_types=[
            pltpu.VMEM((KC,), jnp.int32),
            pltpu.VMEM((KC,), jnp.int32),
            pltpu.VMEM((CH3,), jnp.int32),
            pltpu.VMEM((CH3,), jnp.int32),
            pltpu.VMEM((CAP,), jnp.int32),
            pltpu.VMEM((CAP,), jnp.int32),
            pltpu.VMEM((CAP, H), jnp.float32),
            pltpu.VMEM((KC // NW,), jnp.int32),
            pltpu.VMEM((KC // NW, H), jnp.float32),
            pltpu.VMEM((KC // NW, 16), jnp.float32),
            pltpu.VMEM((N,), jnp.int32),
            pltpu.VMEM_SHARED((ROWS2, H), jnp.float32),
        ],
    )(body)


_k3 = _make_k3()


def _t0_body(col_ref, row_ref, rep_ref):
    col = col_ref[...]
    row = row_ref[...]
    eq = col == row
    jcol = lax.broadcasted_iota(jnp.int32, (KC, KC), 1)
    masked = jnp.where(eq, jcol, jnp.int32(1 << 30))
    rep_ref[...] = jnp.min(masked, axis=1, keepdims=True)


_t0 = pl.pallas_call(
    _t0_body,
    out_shape=jax.ShapeDtypeStruct((KC, 1), jnp.int32),
)


def _t1_body(deg_ref, x_ref, xs_ref, dinv16_ref):
    deg = deg_ref[0, :, 0:1] + deg_ref[1, :, 0:1] + 1.0
    dinv = 1.0 / jnp.sqrt(deg)
    dinv16_ref[...] = jnp.broadcast_to(dinv, (BLK, 16))
    xs = x_ref[...] * dinv
    xs_ref[...] = jnp.concatenate(
        [xs, jnp.zeros((BLK, 16 - DI), jnp.float32)], axis=1)


_t1 = pl.pallas_call(
    _t1_body,
    grid=(GRID,),
    in_specs=[
        pl.BlockSpec((NC, BLK, 16), lambda i: (0, i, 0)),
        pl.BlockSpec((BLK, DI), lambda i: (i, 0)),
    ],
    out_specs=[
        pl.BlockSpec((BLK, 16), lambda i: (i, 0)),
        pl.BlockSpec((BLK, 16), lambda i: (i, 0)),
    ],
    out_shape=[
        jax.ShapeDtypeStruct((N, 16), jnp.float32),
        jax.ShapeDtypeStruct((N, 16), jnp.float32),
    ],
)


def _t2_body(acc_ref, x_ref, dinv16_ref, w1_ref, b1_ref, h1s_ref):
    dinv = dinv16_ref[:, 0:1]
    acc9 = acc_ref[0, :, 0:DI] + acc_ref[1, :, 0:DI]
    agg = acc9 * dinv + x_ref[...] * (dinv * dinv)
    h1 = jnp.dot(agg, w1_ref[...], preferred_element_type=jnp.float32)
    h1 = jnp.maximum(h1 + b1_ref[...], 0.0)
    h1s_ref[...] = h1 * dinv


_t2 = pl.pallas_call(
    _t2_body,
    grid=(GRID,),
    in_specs=[
        pl.BlockSpec((NC, BLK, 16), lambda i: (0, i, 0)),
        pl.BlockSpec((BLK, DI), lambda i: (i, 0)),
        pl.BlockSpec((BLK, 16), lambda i: (i, 0)),
        pl.BlockSpec((DI, H), lambda i: (0, 0)),
        pl.BlockSpec((1, H), lambda i: (0, 0)),
    ],
    out_specs=pl.BlockSpec((BLK, H), lambda i: (i, 0)),
    out_shape=jax.ShapeDtypeStruct((N, H), jnp.float32),
)


def _t3_body(acc2_ref, rep_ref, candrows_ref, canddinv_ref,
             w2_ref, b2_ref, ws1_ref, bs1_ref, ws2_ref, bs2_ref, out_ref):
    total = acc2_ref[0] + acc2_ref[1]
    rep = rep_ref[...]
    jrow = lax.broadcasted_iota(jnp.int32, (KC, ROWS2), 1)
    onehot = jnp.where(rep == jrow, 1.0, 0.0)
    cand_acc = jnp.dot(onehot, total, preferred_element_type=jnp.float32)
    dinv = canddinv_ref[:, 0:1]
    pre = dinv * (cand_acc + candrows_ref[...])
    out2 = jnp.dot(pre, w2_ref[...],
                   preferred_element_type=jnp.float32) + b2_ref[...]
    cand_h = jnp.maximum(out2, 0.0)
    t = jnp.tanh(jnp.dot(cand_h, ws1_ref[...],
                         preferred_element_type=jnp.float32) + bs1_ref[...])
    out_ref[...] = jnp.dot(t, ws2_ref[...],
                           preferred_element_type=jnp.float32) + bs2_ref[...]


_t3 = pl.pallas_call(
    _t3_body,
    out_shape=jax.ShapeDtypeStruct((KC, 1), jnp.float32),
)


def kernel(x, edge_index, candidate_indices, W1, b1, W2, b2, Ws1, bs1, Ws2, bs2):
    cand = candidate_indices.astype(jnp.int32)

    zeros16 = jnp.zeros((N, 16), jnp.float32)
    ones_rows = jnp.ones((CH, 16), jnp.float32)
    negones = jnp.full((N,), -1, jnp.int32)
    zeros2 = jnp.zeros((ROWS2, H), jnp.float32)

    rep = _t0(cand.reshape(KC, 1), cand.reshape(1, KC))

    deg2 = _k1(edge_index, zeros16, ones_rows)
    xs16, dinv16 = _t1(deg2, x)
    acc1 = _k2(edge_index, xs16, zeros16)
    h1s = _t2(acc1, x, dinv16, W1, b1.reshape(1, H))
    acc2, candrows, canddinv = _k3(
        edge_index, cand, rep.reshape(KC), h1s, dinv16, negones, zeros2)
    scores = _t3(acc2, rep, candrows, canddinv,
                 W2, b2.reshape(1, H), Ws1, bs1.reshape(1, H),
                 Ws2, bs2.reshape(1, 1))
    return scores.reshape(KC)
